# baseline (speedup 1.0000x reference)
import functools

import jax
import jax.numpy as jnp
from jax import lax
from jax.experimental import pallas as pl
from jax.experimental.pallas import tpu as pltpu
from jax.experimental.pallas import tpu_sc as plsc

D = 64
SCALE = 8.0
NC = 2
NS = 16
NW = NC * NS
CH = 512
NBUF = 2
ROW_UNROLL = 8


def _scale_chunk(buf):
    def mul_body(i, carry):
        r0 = i * ROW_UNROLL
        for u in range(ROW_UNROLL):
            for c in range(D // 16):
                sl = (r0 + u, pl.ds(c * 16, 16))
                buf[sl] = buf[sl] * SCALE
        return carry
    lax.fori_loop(0, CH // ROW_UNROLL, mul_body, 0)


def _make_sc_gather(B, V):
    b_per_w = B // NW
    nch = b_per_w // CH
    assert b_per_w % CH == 0 and nch >= NBUF and nch % NBUF == 0

    mesh = plsc.VectorSubcoreMesh(core_axis_name="c", subcore_axis_name="s")

    @functools.partial(
        pl.kernel,
        out_type=jax.ShapeDtypeStruct((B, D), jnp.float32),
        mesh=mesh,
        scratch_types=[
            pltpu.VMEM((b_per_w,), jnp.int32),
            [pltpu.VMEM((CH, D), jnp.float32) for _ in range(NBUF)],
            [pltpu.SemaphoreType.DMA for _ in range(NBUF)],
        ],
        compiler_params=pltpu.CompilerParams(use_tc_tiling_on_sc=False),
    )
    def sc_gather(table_hbm, idx_hbm, out_hbm, idx_v, bufs, gsems):
        wid = lax.axis_index("s") * NC + lax.axis_index("c")
        base = pl.multiple_of(wid * b_per_w, b_per_w)
        pltpu.sync_copy(idx_hbm.at[pl.ds(base, b_per_w)], idx_v)

        def idx_slice(g):
            return idx_v.at[pl.ds(pl.multiple_of(g * CH, CH), CH)]

        def start_gather(g, b):
            pltpu.async_copy(table_hbm.at[idx_slice(g)], bufs[b], gsems[b])

        def wait_gather(g, b):
            pltpu.make_async_copy(
                table_hbm.at[idx_slice(g)], bufs[b], gsems[b]).wait()

        def finish_chunk(g, b):
            wait_gather(g, b)
            _scale_chunk(bufs[b])
            row0 = pl.multiple_of(base + g * CH, CH)
            pltpu.sync_copy(bufs[b], out_hbm.at[pl.ds(row0, CH)])

        for b in range(NBUF):
            start_gather(b, b)

        def body(step, carry):
            gbase = step * NBUF
            for b in range(NBUF):
                finish_chunk(gbase + b, b)
                start_gather(gbase + b + NBUF, b)
            return carry
        lax.fori_loop(0, (nch - NBUF) // NBUF, body, 0)

        for b in range(NBUF):
            finish_chunk(nch - NBUF + b, b)

    return sc_gather


def kernel(x, table):
    B0, B1 = x.shape
    V, d = table.shape
    B = B0 * B1
    idx_flat = x.reshape(B)
    out = _make_sc_gather(B, V)(table, idx_flat)
    return out.reshape(B0, B1, d)
